# baseline (speedup 1.0000x reference)
import functools

import jax
import jax.numpy as jnp
from jax import lax
from jax.experimental import pallas as pl
from jax.experimental.pallas import tpu as pltpu
from jax.experimental.pallas import tpu_sc as plsc

L = 16


@functools.partial(jax.jit, static_argnums=(4, 5, 6, 7))
def _sc_gather(tab, idx1, metap, permp, A, V, D, M):
    N = idx1.shape[0]
    SM = metap.shape[0]
    PM = permp.shape[0]
    NC, NS = 2, 16
    NW = NC * NS
    KMAX = SM // NW
    QL = N // 4
    RW = 12800
    NRW = -(-V // RW)

    mesh = plsc.VectorSubcoreMesh(core_axis_name="c", subcore_axis_name="s")

    def extract(ref, j, nch):
        acc = jnp.zeros((L,), jnp.int32)
        for c in range(nch):
            lid = c * L + lax.iota(jnp.int32, L)
            acc = jnp.where(lid == j, ref[pl.ds(c * L, L)], acc)
        return jnp.max(acc)

    def body(tab_hbm, idx1_hbm, meta_hbm, perm_hbm, out_hbm,
             row_v, idx_v, o0_v, o1_v, meta_v, perm_v, sem0, sem1, semr):
        wid = lax.axis_index("s") * NC + lax.axis_index("c")
        pltpu.sync_copy(idx1_hbm, idx_v)
        pltpu.sync_copy(meta_hbm, meta_v)
        pltpu.sync_copy(perm_hbm, perm_v)
        obufs = (o0_v, o1_v)
        sems = (sem0, sem1)

        def slot_body(k, carry):
            g = wid + k * NW
            mg = extract(meta_v, g, SM // L)
            ln = mg >> 16

            @pl.when(ln > 0)
            def _():
                r = mg & 0xFF
                st = (mg >> 8) & 0xFF
                a = r // D
                rdescs = []
                for w in range(NRW):
                    w0 = w * RW
                    wl = min(RW, V - w0)
                    rdescs.append(pltpu.async_copy(
                        tab_hbm.at[pl.ds(r, 1), pl.ds(w0, wl)],
                        row_v.at[:, pl.ds(w0, wl)], semr))
                for d in rdescs:
                    d.wait()

                zeros = jnp.zeros((L,), jnp.int32)

                def drain(b, count):
                    def w_body(j, c2):
                        pltpu.make_async_copy(
                            obufs[b],
                            out_hbm.at[pl.ds(0, 1), pl.ds(0, 1),
                                       pl.ds(0, QL)],
                            sems[b]).wait()
                        return c2
                    lax.fori_loop(0, count, w_body, 0)

                for q in range(4):
                    b = q % 2
                    ob = obufs[b]
                    if q >= 2:
                        drain(b, ln)
                    q0 = q * QL

                    @plsc.parallel_loop(0, QL // L, unroll=4)
                    def _gather(gg, _ob=ob, _q0=q0):
                        iv = idx_v[pl.ds(_q0 + gg * L, L)]
                        v = plsc.load_gather(row_v, [zeros, iv])
                        _ob[0, 0, pl.ds(gg * L, L)] = v

                    def w_body(j, c2):
                        p = extract(perm_v, st + j, PM // L)
                        pltpu.async_copy(
                            ob,
                            out_hbm.at[pl.ds(p, 1), pl.ds(a, 1),
                                       pl.ds(q0, QL)],
                            sems[b])
                        return c2
                    lax.fori_loop(0, ln, w_body, 0)
                drain(0, ln)
                drain(1, ln)
            return carry
        lax.fori_loop(0, KMAX, slot_body, 0)

    run = pl.kernel(
        body,
        out_type=jax.ShapeDtypeStruct((M, A, N), jnp.float32),
        mesh=mesh,
        compiler_params=pltpu.CompilerParams(needs_layout_passes=False),
        scratch_types=[
            pltpu.VMEM((1, V), jnp.float32),
            pltpu.VMEM((N,), jnp.int32),
            pltpu.VMEM((1, 1, QL), jnp.float32),
            pltpu.VMEM((1, 1, QL), jnp.float32),
            pltpu.VMEM((SM,), jnp.int32),
            pltpu.VMEM((PM,), jnp.int32),
            pltpu.SemaphoreType.DMA,
            pltpu.SemaphoreType.DMA,
            pltpu.SemaphoreType.DMA,
        ],
    )
    return run(tab, idx1, metap, permp)


def kernel(x, index1, index2):
    A, V, D = x.shape
    N = index1.shape[0]
    M = index2.shape[0]
    NW = 64
    tab = x.transpose(0, 2, 1).reshape(A * D, V)
    idx1 = index1.reshape(N).astype(jnp.int32)
    i2 = index2.astype(jnp.int32)
    order = jnp.argsort(i2).astype(jnp.int32)
    i2s = i2[order]
    newg = jnp.concatenate(
        [jnp.ones((1,), jnp.int32), (i2s[1:] != i2s[:-1]).astype(jnp.int32)])
    gid = jnp.cumsum(newg) - 1
    pos = jnp.arange(M, dtype=jnp.int32)
    gstart = jax.ops.segment_min(pos, gid, num_segments=M)
    glen = jax.ops.segment_sum(jnp.ones((M,), jnp.int32), gid,
                               num_segments=M)
    gcol = jax.ops.segment_min(i2s, gid, num_segments=M)
    r_ag = jnp.arange(A, dtype=jnp.int32)[:, None] * D + gcol[None, :]
    meta_ag = jnp.where(
        glen[None, :] > 0,
        r_ag | (gstart[None, :] << 8) | (glen[None, :] << 16), 0)
    SM = -(-(A * M) // 32) * 32
    metap = jnp.zeros((SM,), jnp.int32).at[:A * M].set(meta_ag.reshape(A * M))
    pad = (-M) % L
    permp = jnp.concatenate([order, jnp.zeros((pad,), jnp.int32)])
    outP = _sc_gather(tab, idx1, metap, permp, A, V, D, M)
    return outP.transpose(1, 2, 0)


# speedup vs baseline: 1.0884x; 1.0884x over previous
import functools

import jax
import jax.numpy as jnp
from jax import lax
from jax.experimental import pallas as pl
from jax.experimental.pallas import tpu as pltpu
from jax.experimental.pallas import tpu_sc as plsc

L = 16


@functools.partial(jax.jit, static_argnums=(4, 5, 6, 7))
def _sc_gather(tab, idx1, metap, permp, A, V, D, M):
    N = idx1.shape[0]
    SM = metap.shape[0]
    PM = permp.shape[0]
    NC, NS = 2, 16
    NW = NC * NS
    KMAX = SM // NW
    QL = N // 4
    RW = 12800
    NRW = -(-V // RW)

    mesh = plsc.VectorSubcoreMesh(core_axis_name="c", subcore_axis_name="s")

    def extract(ref, j, nch):
        acc = jnp.zeros((L,), jnp.int32)
        for c in range(nch):
            lid = c * L + lax.iota(jnp.int32, L)
            acc = jnp.where(lid == j, ref[pl.ds(c * L, L)], acc)
        return jnp.max(acc)

    DOM = (A // NC) * M

    def body(tab_hbm, idx1_hbm, meta_hbm, perm_hbm, out_hbm,
             row_v, idx_v, o0_v, o1_v, meta_v, perm_v, cnt_s,
             sem0, sem1, semr):
        cid = lax.axis_index("c")
        sid = lax.axis_index("s")
        pltpu.sync_copy(idx1_hbm, idx_v)
        pltpu.sync_copy(meta_hbm, meta_v)
        pltpu.sync_copy(perm_hbm, perm_v)
        obufs = (o0_v, o1_v)
        sems = (sem0, sem1)

        @pl.when(sid == 0)
        def _():
            cnt_s[0] = 0
        plsc.subcore_barrier()

        def slot_body(i, carry):
            g = cid * DOM + i
            mg = extract(meta_v, g, SM // L)
            ln = mg >> 16

            @pl.when(ln > 0)
            def _():
                r = mg & 0xFF
                st = (mg >> 8) & 0xFF
                a = r // D
                rdescs = []
                for w in range(NRW):
                    w0 = w * RW
                    wl = min(RW, V - w0)
                    rdescs.append(pltpu.async_copy(
                        tab_hbm.at[pl.ds(r, 1), pl.ds(w0, wl)],
                        row_v.at[:, pl.ds(w0, wl)], semr))
                for d in rdescs:
                    d.wait()

                zeros = jnp.zeros((L,), jnp.int32)

                def drain(b, count):
                    def w_body(j, c2):
                        pltpu.make_async_copy(
                            obufs[b],
                            out_hbm.at[pl.ds(0, 1), pl.ds(0, 1),
                                       pl.ds(0, QL)],
                            sems[b]).wait()
                        return c2
                    lax.fori_loop(0, count, w_body, 0)

                for q in range(4):
                    b = q % 2
                    ob = obufs[b]
                    if q >= 2:
                        drain(b, ln)
                    q0 = q * QL

                    @plsc.parallel_loop(0, QL // L, unroll=4)
                    def _gather(gg, _ob=ob, _q0=q0):
                        iv = idx_v[pl.ds(_q0 + gg * L, L)]
                        v = plsc.load_gather(row_v, [zeros, iv])
                        _ob[0, 0, pl.ds(gg * L, L)] = v

                    def w_body(j, c2):
                        p = extract(perm_v, st + j, PM // L)
                        pltpu.async_copy(
                            ob,
                            out_hbm.at[pl.ds(p, 1), pl.ds(a, 1),
                                       pl.ds(q0, QL)],
                            sems[b])
                        return c2
                    lax.fori_loop(0, ln, w_body, 0)
                drain(0, ln)
                drain(1, ln)
            return carry

        def steal_body(i):
            slot_body(i, 0)
            return plsc.fetch_and_add(cnt_s.at[0], 1, subcore_id=0)
        i0 = plsc.fetch_and_add(cnt_s.at[0], 1, subcore_id=0)
        lax.while_loop(lambda i: i < DOM, steal_body, i0)

    run = pl.kernel(
        body,
        out_type=jax.ShapeDtypeStruct((M, A, N), jnp.float32),
        mesh=mesh,
        compiler_params=pltpu.CompilerParams(needs_layout_passes=False),
        scratch_types=[
            pltpu.VMEM((1, V), jnp.float32),
            pltpu.VMEM((N,), jnp.int32),
            pltpu.VMEM((1, 1, QL), jnp.float32),
            pltpu.VMEM((1, 1, QL), jnp.float32),
            pltpu.VMEM((SM,), jnp.int32),
            pltpu.VMEM((PM,), jnp.int32),
            pltpu.SMEM((8,), jnp.int32),
            pltpu.SemaphoreType.DMA,
            pltpu.SemaphoreType.DMA,
            pltpu.SemaphoreType.DMA,
        ],
    )
    return run(tab, idx1, metap, permp)


def kernel(x, index1, index2):
    A, V, D = x.shape
    N = index1.shape[0]
    M = index2.shape[0]
    NW = 64
    tab = x.transpose(0, 2, 1).reshape(A * D, V)
    idx1 = index1.reshape(N).astype(jnp.int32)
    i2 = index2.astype(jnp.int32)
    order = jnp.argsort(i2).astype(jnp.int32)
    i2s = i2[order]
    newg = jnp.concatenate(
        [jnp.ones((1,), jnp.int32), (i2s[1:] != i2s[:-1]).astype(jnp.int32)])
    gid = jnp.cumsum(newg) - 1
    pos = jnp.arange(M, dtype=jnp.int32)
    gstart = jax.ops.segment_min(pos, gid, num_segments=M)
    glen = jax.ops.segment_sum(jnp.ones((M,), jnp.int32), gid,
                               num_segments=M)
    gcol = jax.ops.segment_min(i2s, gid, num_segments=M)
    r_ag = jnp.arange(A, dtype=jnp.int32)[:, None] * D + gcol[None, :]
    meta_ag = jnp.where(
        glen[None, :] > 0,
        r_ag | (gstart[None, :] << 8) | (glen[None, :] << 16), 0)
    SM = -(-(A * M) // 32) * 32
    metap = jnp.zeros((SM,), jnp.int32).at[:A * M].set(meta_ag.reshape(A * M))
    pad = (-M) % L
    permp = jnp.concatenate([order, jnp.zeros((pad,), jnp.int32)])
    outP = _sc_gather(tab, idx1, metap, permp, A, V, D, M)
    return outP.transpose(1, 2, 0)


# speedup vs baseline: 1.1050x; 1.0153x over previous
import functools

import jax
import jax.numpy as jnp
from jax import lax
from jax.experimental import pallas as pl
from jax.experimental.pallas import tpu as pltpu
from jax.experimental.pallas import tpu_sc as plsc

L = 16


@functools.partial(jax.jit, static_argnums=(4, 5, 6, 7))
def _sc_gather(tab, idx1, metap, permp, A, V, D, M):
    N = idx1.shape[0]
    SM = metap.shape[0]
    PM = permp.shape[0]
    NC, NS = 2, 16
    NW = NC * NS
    KMAX = SM // NW
    QL = N // 4
    RW = 12800
    NRW = -(-V // RW)

    mesh = plsc.VectorSubcoreMesh(core_axis_name="c", subcore_axis_name="s")

    def extract(ref, j, nch):
        del nch
        j0 = (j // L) * L
        lid = lax.iota(jnp.int32, L)
        acc = jnp.where(lid == j - j0, ref[pl.ds(j0, L)], 0)
        return jnp.max(acc)

    DOM = (A // NC) * M

    def body(tab_hbm, idx1_hbm, meta_hbm, perm_hbm, out_hbm,
             row_v, idx_v, o0_v, o1_v, meta_v, perm_v, cnt_s,
             sem0, sem1, semr):
        cid = lax.axis_index("c")
        sid = lax.axis_index("s")
        pltpu.sync_copy(idx1_hbm, idx_v)
        pltpu.sync_copy(meta_hbm, meta_v)
        pltpu.sync_copy(perm_hbm, perm_v)
        obufs = (o0_v, o1_v)
        sems = (sem0, sem1)

        @pl.when(sid == 0)
        def _():
            cnt_s[0] = 0
        plsc.subcore_barrier()

        def slot_body(i, carry):
            g = cid * DOM + i
            mg = extract(meta_v, g, SM // L)
            ln = mg >> 16

            @pl.when(ln > 0)
            def _():
                r = mg & 0xFF
                st = (mg >> 8) & 0xFF
                a = r // D
                rdescs = []
                for w in range(NRW):
                    w0 = w * RW
                    wl = min(RW, V - w0)
                    rdescs.append(pltpu.async_copy(
                        tab_hbm.at[pl.ds(r, 1), pl.ds(w0, wl)],
                        row_v.at[:, pl.ds(w0, wl)], semr))
                for d in rdescs:
                    d.wait()

                zeros = jnp.zeros((L,), jnp.int32)

                def drain(b, count):
                    def w_body(j, c2):
                        pltpu.make_async_copy(
                            obufs[b],
                            out_hbm.at[pl.ds(0, 1), pl.ds(0, 1),
                                       pl.ds(0, QL)],
                            sems[b]).wait()
                        return c2
                    lax.fori_loop(0, count, w_body, 0)

                for q in range(4):
                    b = q % 2
                    ob = obufs[b]
                    if q >= 2:
                        drain(b, ln)
                    q0 = q * QL

                    @plsc.parallel_loop(0, QL // L, unroll=8)
                    def _gather(gg, _ob=ob, _q0=q0):
                        iv = idx_v[pl.ds(_q0 + gg * L, L)]
                        v = plsc.load_gather(row_v, [zeros, iv])
                        _ob[0, 0, pl.ds(gg * L, L)] = v

                    def w_body(j, c2):
                        p = extract(perm_v, st + j, PM // L)
                        pltpu.async_copy(
                            ob,
                            out_hbm.at[pl.ds(p, 1), pl.ds(a, 1),
                                       pl.ds(q0, QL)],
                            sems[b])
                        return c2
                    lax.fori_loop(0, ln, w_body, 0)
                drain(0, ln)
                drain(1, ln)
            return carry

        def steal_body(i):
            slot_body(i, 0)
            return plsc.fetch_and_add(cnt_s.at[0], 1, subcore_id=0)
        i0 = plsc.fetch_and_add(cnt_s.at[0], 1, subcore_id=0)
        lax.while_loop(lambda i: i < DOM, steal_body, i0)

    run = pl.kernel(
        body,
        out_type=jax.ShapeDtypeStruct((M, A, N), jnp.float32),
        mesh=mesh,
        compiler_params=pltpu.CompilerParams(needs_layout_passes=False),
        scratch_types=[
            pltpu.VMEM((1, V), jnp.float32),
            pltpu.VMEM((N,), jnp.int32),
            pltpu.VMEM((1, 1, QL), jnp.float32),
            pltpu.VMEM((1, 1, QL), jnp.float32),
            pltpu.VMEM((SM,), jnp.int32),
            pltpu.VMEM((PM,), jnp.int32),
            pltpu.SMEM((8,), jnp.int32),
            pltpu.SemaphoreType.DMA,
            pltpu.SemaphoreType.DMA,
            pltpu.SemaphoreType.DMA,
        ],
    )
    return run(tab, idx1, metap, permp)


def kernel(x, index1, index2):
    A, V, D = x.shape
    N = index1.shape[0]
    M = index2.shape[0]
    NW = 64
    tab = x.transpose(0, 2, 1).reshape(A * D, V)
    idx1 = index1.reshape(N).astype(jnp.int32)
    i2 = index2.astype(jnp.int32)
    order = jnp.argsort(i2).astype(jnp.int32)
    i2s = i2[order]
    newg = jnp.concatenate(
        [jnp.ones((1,), jnp.int32), (i2s[1:] != i2s[:-1]).astype(jnp.int32)])
    gid = jnp.cumsum(newg) - 1
    pos = jnp.arange(M, dtype=jnp.int32)
    gstart = jax.ops.segment_min(pos, gid, num_segments=M)
    glen = jax.ops.segment_sum(jnp.ones((M,), jnp.int32), gid,
                               num_segments=M)
    gcol = jax.ops.segment_min(i2s, gid, num_segments=M)
    r_ag = jnp.arange(A, dtype=jnp.int32)[:, None] * D + gcol[None, :]
    meta_ag = jnp.where(
        glen[None, :] > 0,
        r_ag | (gstart[None, :] << 8) | (glen[None, :] << 16), 0)
    SM = -(-(A * M) // 32) * 32
    metap = jnp.zeros((SM,), jnp.int32).at[:A * M].set(meta_ag.reshape(A * M))
    pad = (-M) % L
    permp = jnp.concatenate([order, jnp.zeros((pad,), jnp.int32)])
    outP = _sc_gather(tab, idx1, metap, permp, A, V, D, M)
    return outP.transpose(1, 2, 0)
